# baseline (speedup 1.0000x reference)
import jax
import jax.numpy as jnp
from jax.experimental import pallas as pl
from jax.experimental.pallas import tpu as pltpu


def _dot_t(a, b):
    return jax.lax.dot_general(a, b, (((1,), (1,)), ((), ())),
                               preferred_element_type=jnp.float32)


def _dot_kk(a, b):
    return jax.lax.dot_general(a, b, (((0,), (0,)), ((), ())),
                               preferred_element_type=jnp.float32)


def _proj_kernel(dt_ref, maskf_ref, rl_ref, base_ref,
                 ce_ref, re_ref, dwrow_ref, db_ref,
                 wb_ref, wc_ref, wr_ref, wd_ref, pb_ref,
                 out_ref):
    _, G, Nn, Dd = base_ref.shape
    R = G * Nn
    n_run = re_ref.shape[0]
    n_tab = 2 * n_run

    cc = _dot_t(ce_ref[...], wc_ref[...])
    rr = _dot_t(re_ref[...], wr_ref[...])
    vv = _dot_t(dwrow_ref[...], wd_ref[...])
    const = _dot_t(db_ref[...], wd_ref[...]) + pb_ref[...]
    table = jnp.concatenate([rr + cc[0:1, :] + const,
                             rr + cc[1:2, :] + const], axis=0)

    x = base_ref[...].reshape(R, Dd)
    mm = _dot_t(x, wb_ref[...]).reshape(G, Nn, Dd)

    idxf = (maskf_ref[...].reshape(G, Nn) * n_run
            + jnp.clip(rl_ref[...].reshape(G, Nn), 0, n_run - 1)
            .astype(jnp.float32))

    kio = jax.lax.broadcasted_iota(jnp.int32, (n_tab, Nn), 0).astype(
        jnp.float32)
    for g in range(G):
        idx_row = idxf[g:g + 1, :]
        oh_t = (kio == idx_row).astype(jnp.float32)
        lk = _dot_kk(oh_t, table)
        out_ref[0, g, :, :] = mm[g] + lk + dt_ref[0, g, 0] * vv


def kernel(base, change_mask, run_length, delta_t, change_embed, run_embed,
           delta_W, delta_b, proj_W, proj_b):
    Bb, Tt, Nn, Dd = base.shape
    Ee = change_embed.shape[1]
    BT = Bb * Tt
    G = 8
    grid = (Bb, Tt // G)

    maskf = change_mask.astype(jnp.float32)
    rl2 = run_length.astype(jnp.int32)
    dt3 = delta_t.astype(jnp.float32).reshape(Bb, Tt, 1)
    wb = proj_W[:, :Dd]
    wc = proj_W[:, Dd:Dd + Ee]
    wr = proj_W[:, Dd + Ee:Dd + 2 * Ee]
    wd = proj_W[:, Dd + 2 * Ee:Dd + 3 * Ee]
    dwrow = delta_W.reshape(1, Ee)
    db2 = delta_b.reshape(1, Ee)
    pb2 = proj_b.reshape(1, Dd)

    rep = lambda shape: pl.BlockSpec(shape, lambda b, t: (0, 0))
    out = pl.pallas_call(
        _proj_kernel,
        grid=grid,
        in_specs=[
            pl.BlockSpec((1, G, 1), lambda b, t: (b, t, 0)),
            pl.BlockSpec((1, G, Nn), lambda b, t: (b, t, 0)),
            pl.BlockSpec((1, G, Nn), lambda b, t: (b, t, 0)),
            pl.BlockSpec((1, G, Nn, Dd), lambda b, t: (b, t, 0, 0)),
            rep(change_embed.shape),
            rep(run_embed.shape),
            rep((1, Ee)),
            rep((1, Ee)),
            rep((Dd, Dd)),
            rep((Dd, Ee)),
            rep((Dd, Ee)),
            rep((Dd, Ee)),
            rep((1, Dd)),
        ],
        out_specs=pl.BlockSpec((1, G, Nn, Dd), lambda b, t: (b, t, 0, 0)),
        out_shape=jax.ShapeDtypeStruct((Bb, Tt, Nn, Dd), jnp.float32),
        compiler_params=pltpu.CompilerParams(
            dimension_semantics=("parallel", "parallel")),
    )(dt3, maskf, rl2, base, change_embed, run_embed, dwrow, db2,
      wb, wc, wr, wd, pb2)
    return out


# speedup vs baseline: 1.0021x; 1.0021x over previous
import jax
import jax.numpy as jnp
from jax.experimental import pallas as pl
from jax.experimental.pallas import tpu as pltpu


def _dot_t(a, b):
    return jax.lax.dot_general(a, b, (((1,), (1,)), ((), ())),
                               preferred_element_type=jnp.float32)


def _dot_kk(a, b):
    return jax.lax.dot_general(a, b, (((0,), (0,)), ((), ())),
                               preferred_element_type=jnp.float32)


def _proj_kernel(dt_ref, maskf_ref, rl_ref, base_ref,
                 ce_ref, re_ref, dwrow_ref, db_ref,
                 wb_ref, wc_ref, wr_ref, wd_ref, pb_ref,
                 out_ref):
    _, G, Nn, Dd = base_ref.shape
    R = G * Nn
    n_run = re_ref.shape[0]
    n_tab = 2 * n_run

    cc = _dot_t(ce_ref[...], wc_ref[...])
    rr = _dot_t(re_ref[...], wr_ref[...])
    vv = _dot_t(dwrow_ref[...], wd_ref[...])
    const = _dot_t(db_ref[...], wd_ref[...]) + pb_ref[...]
    table = jnp.concatenate([rr + cc[0:1, :] + const,
                             rr + cc[1:2, :] + const], axis=0)

    x = base_ref[...].reshape(R, Dd)
    mm = _dot_t(x, wb_ref[...]).reshape(G, Nn, Dd)

    idxf = (maskf_ref[...].reshape(G, Nn) * n_run
            + jnp.clip(rl_ref[...].reshape(G, Nn), 0, n_run - 1)
            .astype(jnp.float32))

    kio = jax.lax.broadcasted_iota(jnp.int32, (n_tab, Nn), 0).astype(
        jnp.float32)
    for g in range(G):
        idx_row = idxf[g:g + 1, :]
        oh_t = (kio == idx_row).astype(jnp.float32)
        lk = _dot_kk(oh_t, table)
        out_ref[0, g, :, :] = mm[g] + lk + dt_ref[0, g, 0] * vv


def kernel(base, change_mask, run_length, delta_t, change_embed, run_embed,
           delta_W, delta_b, proj_W, proj_b):
    Bb, Tt, Nn, Dd = base.shape
    Ee = change_embed.shape[1]
    BT = Bb * Tt
    G = 8
    TG = Tt // G
    grid = (Bb * TG,)

    maskf = change_mask.astype(jnp.float32)
    rl2 = run_length.astype(jnp.int32)
    dt3 = delta_t.astype(jnp.float32).reshape(Bb, Tt, 1)
    wb = proj_W[:, :Dd]
    wc = proj_W[:, Dd:Dd + Ee]
    wr = proj_W[:, Dd + Ee:Dd + 2 * Ee]
    wd = proj_W[:, Dd + 2 * Ee:Dd + 3 * Ee]
    dwrow = delta_W.reshape(1, Ee)
    db2 = delta_b.reshape(1, Ee)
    pb2 = proj_b.reshape(1, Dd)

    rep = lambda shape: pl.BlockSpec(shape, lambda i: (0, 0))
    out = pl.pallas_call(
        _proj_kernel,
        grid=grid,
        in_specs=[
            pl.BlockSpec((1, G, 1), lambda i: (i // TG, i % TG, 0)),
            pl.BlockSpec((1, G, Nn), lambda i: (i // TG, i % TG, 0)),
            pl.BlockSpec((1, G, Nn), lambda i: (i // TG, i % TG, 0)),
            pl.BlockSpec((1, G, Nn, Dd),
                         lambda i: (i // TG, i % TG, 0, 0)),
            rep(change_embed.shape),
            rep(run_embed.shape),
            rep((1, Ee)),
            rep((1, Ee)),
            rep((Dd, Dd)),
            rep((Dd, Ee)),
            rep((Dd, Ee)),
            rep((Dd, Ee)),
            rep((1, Dd)),
        ],
        out_specs=pl.BlockSpec((1, G, Nn, Dd),
                               lambda i: (i // TG, i % TG, 0, 0)),
        out_shape=jax.ShapeDtypeStruct((Bb, Tt, Nn, Dd), jnp.float32),
        compiler_params=pltpu.CompilerParams(
            dimension_semantics=("parallel",)),
    )(dt3, maskf, rl2, base, change_embed, run_embed, dwrow, db2,
      wb, wc, wr, wd, pb2)
    return out


# speedup vs baseline: 2.7314x; 2.7258x over previous
import jax
import jax.numpy as jnp
from jax.experimental import pallas as pl
from jax.experimental.pallas import tpu as pltpu


def _dot_t(a, b):
    return jax.lax.dot_general(a, b, (((1,), (1,)), ((), ())),
                               preferred_element_type=jnp.float32)


def _dot(a, b):
    return jax.lax.dot_general(a, b, (((1,), (0,)), ((), ())),
                               preferred_element_type=jnp.float32)


def _proj_kernel(dt_ref, maskf_ref, rl_ref, base_ref,
                 ce_ref, re_ref, dwx_ref, dbx_ref,
                 wb_ref, wall_ref,
                 out_ref):
    _, G, Dd, Nn = base_ref.shape
    n_run = re_ref.shape[0]
    Ee = ce_ref.shape[1]

    z = lambda r, c: jnp.zeros((r, c), jnp.float32)
    re = re_ref[...]
    dce = ce_ref[1:2, :] - ce_ref[0:1, :]
    rhs = jnp.concatenate([
        jnp.concatenate([z(n_run, Ee), re, z(n_run, Ee + 1)], axis=1),
        jnp.concatenate([dce, z(1, 2 * Ee + 1)], axis=1),
        jnp.concatenate([ce_ref[0:1, :], z(1, Ee), dbx_ref[...]], axis=1),
        jnp.concatenate([z(1, 2 * Ee), dwx_ref[...]], axis=1),
    ], axis=0)
    table = _dot_t(wall_ref[...], rhs)

    idxf = jnp.clip(rl_ref[0], 0, n_run - 1).astype(jnp.float32)
    kio = jax.lax.broadcasted_iota(jnp.int32, (n_run, Nn), 0).astype(
        jnp.float32)
    ones_row = jnp.ones((1, Nn), jnp.float32)
    wb = wb_ref[...]
    for g in range(G):
        oh = (kio == idxf[g:g + 1, :]).astype(jnp.float32)
        dt_row = jnp.broadcast_to(dt_ref[0, g, 0], (1, Nn))
        oh_aug = jnp.concatenate(
            [oh, maskf_ref[0, g:g + 1, :], ones_row, dt_row], axis=0)
        xt = base_ref[0, g]
        out_ref[0, g] = _dot(wb, xt) + _dot(table, oh_aug)


def kernel(base, change_mask, run_length, delta_t, change_embed, run_embed,
           delta_W, delta_b, proj_W, proj_b):
    Bb, Tt, Nn, Dd = base.shape
    Ee = change_embed.shape[1]
    G = 8
    TG = Tt // G
    grid = (Bb * TG,)

    base_t = jnp.transpose(base, (0, 1, 3, 2))
    maskf = change_mask.astype(jnp.float32)
    rl2 = run_length.astype(jnp.int32)
    dt3 = delta_t.astype(jnp.float32).reshape(Bb, Tt, 1)
    wb = proj_W[:, :Dd]
    wc = proj_W[:, Dd:Dd + Ee]
    wr = proj_W[:, Dd + Ee:Dd + 2 * Ee]
    wd = proj_W[:, Dd + 2 * Ee:Dd + 3 * Ee]
    wall = jnp.concatenate([wc, wr, wd, proj_b.reshape(Dd, 1)], axis=1)
    dwx = jnp.concatenate([delta_W.reshape(1, Ee),
                           jnp.zeros((1, 1), jnp.float32)], axis=1)
    dbx = jnp.concatenate([delta_b.reshape(1, Ee),
                           jnp.ones((1, 1), jnp.float32)], axis=1)

    rep = lambda shape: pl.BlockSpec(shape, lambda i: (0, 0))
    out_t = pl.pallas_call(
        _proj_kernel,
        grid=grid,
        in_specs=[
            pl.BlockSpec((1, G, 1), lambda i: (i // TG, i % TG, 0)),
            pl.BlockSpec((1, G, Nn), lambda i: (i // TG, i % TG, 0)),
            pl.BlockSpec((1, G, Nn), lambda i: (i // TG, i % TG, 0)),
            pl.BlockSpec((1, G, Dd, Nn),
                         lambda i: (i // TG, i % TG, 0, 0)),
            rep(change_embed.shape),
            rep(run_embed.shape),
            rep((1, Ee + 1)),
            rep((1, Ee + 1)),
            rep((Dd, Dd)),
            rep((Dd, 3 * Ee + 1)),
        ],
        out_specs=pl.BlockSpec((1, G, Dd, Nn),
                               lambda i: (i // TG, i % TG, 0, 0)),
        out_shape=jax.ShapeDtypeStruct((Bb, Tt, Dd, Nn), jnp.float32),
        compiler_params=pltpu.CompilerParams(
            dimension_semantics=("parallel",)),
    )(dt3, maskf, rl2, base_t, change_embed, run_embed, dwx, dbx,
      wb, wall)
    return jnp.transpose(out_t, (0, 1, 3, 2))


# speedup vs baseline: 6.0079x; 2.1996x over previous
import jax
import jax.numpy as jnp
from jax.experimental import pallas as pl
from jax.experimental.pallas import tpu as pltpu


def _dot_t(a, b):
    return jax.lax.dot_general(a, b, (((1,), (1,)), ((), ())),
                               preferred_element_type=jnp.float32)


def _dot(a, b):
    return jax.lax.dot_general(a, b, (((1,), (0,)), ((), ())),
                               preferred_element_type=jnp.float32)


def _proj_kernel(dt_ref, maskf_ref, rl_ref, base_ref,
                 ce_ref, re_ref, dwx_ref, dbx_ref,
                 wb_ref, wall_ref,
                 out_ref):
    _, G, Dd, Nn = base_ref.shape
    n_run = re_ref.shape[0]
    Ee = ce_ref.shape[1]

    z = lambda r, c: jnp.zeros((r, c), jnp.float32)
    re = re_ref[...]
    dce = ce_ref[1:2, :] - ce_ref[0:1, :]
    rhs = jnp.concatenate([
        jnp.concatenate([z(n_run, Ee), re, z(n_run, Ee + 1)], axis=1),
        jnp.concatenate([dce, z(1, 2 * Ee + 1)], axis=1),
        jnp.concatenate([ce_ref[0:1, :], z(1, Ee), dbx_ref[...]], axis=1),
        jnp.concatenate([z(1, 2 * Ee), dwx_ref[...]], axis=1),
    ], axis=0)
    table = _dot_t(wall_ref[...], rhs)

    idxf = jnp.clip(rl_ref[0], 0, n_run - 1).astype(jnp.float32)
    kio = jax.lax.broadcasted_iota(jnp.int32, (n_run, Nn), 0).astype(
        jnp.float32)
    ones_row = jnp.ones((1, Nn), jnp.float32)
    wb = wb_ref[...]
    for g in range(G):
        oh = (kio == idxf[g:g + 1, :]).astype(jnp.float32)
        dt_row = jnp.broadcast_to(dt_ref[0, g, 0], (1, Nn))
        oh_aug = jnp.concatenate(
            [oh, maskf_ref[0, g:g + 1, :], ones_row, dt_row], axis=0)
        xt = base_ref[0, g]
        out_ref[0, g] = _dot(wb, xt) + _dot(table, oh_aug)


def kernel(base, change_mask, run_length, delta_t, change_embed, run_embed,
           delta_W, delta_b, proj_W, proj_b):
    Bb, Tt, Nn, Dd = base.shape
    Ee = change_embed.shape[1]
    G = 40
    TG = Tt // G
    grid = (Bb * TG,)

    base_t = jnp.transpose(base, (0, 1, 3, 2))
    maskf = change_mask.astype(jnp.float32)
    rl2 = run_length.astype(jnp.int32)
    dt3 = delta_t.astype(jnp.float32).reshape(Bb, Tt, 1)
    wb = proj_W[:, :Dd]
    wc = proj_W[:, Dd:Dd + Ee]
    wr = proj_W[:, Dd + Ee:Dd + 2 * Ee]
    wd = proj_W[:, Dd + 2 * Ee:Dd + 3 * Ee]
    wall = jnp.concatenate([wc, wr, wd, proj_b.reshape(Dd, 1)], axis=1)
    dwx = jnp.concatenate([delta_W.reshape(1, Ee),
                           jnp.zeros((1, 1), jnp.float32)], axis=1)
    dbx = jnp.concatenate([delta_b.reshape(1, Ee),
                           jnp.ones((1, 1), jnp.float32)], axis=1)

    rep = lambda shape: pl.BlockSpec(shape, lambda i: (0, 0))
    out_t = pl.pallas_call(
        _proj_kernel,
        grid=grid,
        in_specs=[
            pl.BlockSpec((1, G, 1), lambda i: (i // TG, i % TG, 0)),
            pl.BlockSpec((1, G, Nn), lambda i: (i // TG, i % TG, 0)),
            pl.BlockSpec((1, G, Nn), lambda i: (i // TG, i % TG, 0)),
            pl.BlockSpec((1, G, Dd, Nn),
                         lambda i: (i // TG, i % TG, 0, 0)),
            rep(change_embed.shape),
            rep(run_embed.shape),
            rep((1, Ee + 1)),
            rep((1, Ee + 1)),
            rep((Dd, Dd)),
            rep((Dd, 3 * Ee + 1)),
        ],
        out_specs=pl.BlockSpec((1, G, Dd, Nn),
                               lambda i: (i // TG, i % TG, 0, 0)),
        out_shape=jax.ShapeDtypeStruct((Bb, Tt, Dd, Nn), jnp.float32),
        compiler_params=pltpu.CompilerParams(
            dimension_semantics=("parallel",)),
    )(dt3, maskf, rl2, base_t, change_embed, run_embed, dwx, dbx,
      wb, wall)
    return jnp.transpose(out_t, (0, 1, 3, 2))


# speedup vs baseline: 6.9824x; 1.1622x over previous
import jax
import jax.numpy as jnp
from jax.experimental import pallas as pl
from jax.experimental.pallas import tpu as pltpu


def _dot_t(a, b):
    return jax.lax.dot_general(a, b, (((1,), (1,)), ((), ())),
                               preferred_element_type=jnp.float32)


def _dot(a, b):
    return jax.lax.dot_general(a, b, (((1,), (0,)), ((), ())),
                               preferred_element_type=jnp.float32)


def _proj_kernel(dt_ref, maskf_ref, rl_ref, base_ref,
                 ce_ref, re_ref, dwx_ref, dbx_ref,
                 wb_ref, wall_ref,
                 out_ref):
    _, G, Dd, Nn = base_ref.shape
    n_run = re_ref.shape[0]
    Ee = ce_ref.shape[1]

    z = lambda r, c: jnp.zeros((r, c), jnp.float32)
    re = re_ref[...]
    dce = ce_ref[1:2, :] - ce_ref[0:1, :]
    rhs = jnp.concatenate([
        jnp.concatenate([z(n_run, Ee), re, z(n_run, Ee + 1)], axis=1),
        jnp.concatenate([dce, z(1, 2 * Ee + 1)], axis=1),
        jnp.concatenate([ce_ref[0:1, :], z(1, Ee), dbx_ref[...]], axis=1),
        jnp.concatenate([z(1, 2 * Ee), dwx_ref[...]], axis=1),
    ], axis=0)
    table = _dot_t(wall_ref[...], rhs)

    idxf = jnp.clip(rl_ref[0], 0, n_run - 1).astype(jnp.float32)
    kio = jax.lax.broadcasted_iota(jnp.int32, (n_run, Nn), 0).astype(
        jnp.float32)
    ones_row = jnp.ones((1, Nn), jnp.float32)
    wb = wb_ref[...]
    for g in range(G):
        oh = (kio == idxf[g:g + 1, :]).astype(jnp.float32)
        dt_row = jnp.broadcast_to(dt_ref[0, 0, g], (1, Nn))
        oh_aug = jnp.concatenate(
            [oh, maskf_ref[0, g:g + 1, :], ones_row, dt_row], axis=0)
        xt = base_ref[0, g]
        out_ref[0, g] = _dot(wb, xt) + _dot(table, oh_aug)


def kernel(base, change_mask, run_length, delta_t, change_embed, run_embed,
           delta_W, delta_b, proj_W, proj_b):
    Bb, Tt, Nn, Dd = base.shape
    Ee = change_embed.shape[1]
    G = 100
    TG = Tt // G
    grid = (Bb * TG,)

    base_t = jnp.transpose(base, (0, 1, 3, 2))
    NB = Bb * TG
    maskf = change_mask.astype(jnp.float32).reshape(NB, G, Nn)
    rl2 = run_length.astype(jnp.int32).reshape(NB, G, Nn)
    dt3 = delta_t.astype(jnp.float32).reshape(NB, 1, G)
    wb = proj_W[:, :Dd]
    wc = proj_W[:, Dd:Dd + Ee]
    wr = proj_W[:, Dd + Ee:Dd + 2 * Ee]
    wd = proj_W[:, Dd + 2 * Ee:Dd + 3 * Ee]
    wall = jnp.concatenate([wc, wr, wd, proj_b.reshape(Dd, 1)], axis=1)
    dwx = jnp.concatenate([delta_W.reshape(1, Ee),
                           jnp.zeros((1, 1), jnp.float32)], axis=1)
    dbx = jnp.concatenate([delta_b.reshape(1, Ee),
                           jnp.ones((1, 1), jnp.float32)], axis=1)

    rep = lambda shape: pl.BlockSpec(shape, lambda i: (0, 0))
    out_t = pl.pallas_call(
        _proj_kernel,
        grid=grid,
        in_specs=[
            pl.BlockSpec((1, 1, G), lambda i: (i, 0, 0)),
            pl.BlockSpec((1, G, Nn), lambda i: (i, 0, 0)),
            pl.BlockSpec((1, G, Nn), lambda i: (i, 0, 0)),
            pl.BlockSpec((1, G, Dd, Nn),
                         lambda i: (i // TG, i % TG, 0, 0)),
            rep(change_embed.shape),
            rep(run_embed.shape),
            rep((1, Ee + 1)),
            rep((1, Ee + 1)),
            rep((Dd, Dd)),
            rep((Dd, 3 * Ee + 1)),
        ],
        out_specs=pl.BlockSpec((1, G, Dd, Nn),
                               lambda i: (i // TG, i % TG, 0, 0)),
        out_shape=jax.ShapeDtypeStruct((Bb, Tt, Dd, Nn), jnp.float32),
        compiler_params=pltpu.CompilerParams(
            dimension_semantics=("parallel",)),
    )(dt3, maskf, rl2, base_t, change_embed, run_embed, dwx, dbx,
      wb, wall)
    return jnp.transpose(out_t, (0, 1, 3, 2))


# speedup vs baseline: 7.2555x; 1.0391x over previous
import jax
import jax.numpy as jnp
from jax.experimental import pallas as pl
from jax.experimental.pallas import tpu as pltpu


def _dot_t(a, b):
    return jax.lax.dot_general(a, b, (((1,), (1,)), ((), ())),
                               preferred_element_type=jnp.float32)


def _dot(a, b):
    return jax.lax.dot_general(a, b, (((1,), (0,)), ((), ())),
                               preferred_element_type=jnp.float32)


def _proj_kernel(dt_ref, maskf_ref, rl_ref, base_ref,
                 ce_ref, re_ref, dwx_ref, dbx_ref,
                 wb_ref, wall_ref,
                 out_ref):
    _, G, Dd, Nn = base_ref.shape
    n_run = re_ref.shape[0]
    Ee = ce_ref.shape[1]

    z = lambda r, c: jnp.zeros((r, c), jnp.float32)
    re = re_ref[...]
    dce = ce_ref[1:2, :] - ce_ref[0:1, :]
    rhs = jnp.concatenate([
        jnp.concatenate([z(n_run, Ee), re, z(n_run, Ee + 1)], axis=1),
        jnp.concatenate([dce, z(1, 2 * Ee + 1)], axis=1),
        jnp.concatenate([ce_ref[0:1, :], z(1, Ee), dbx_ref[...]], axis=1),
        jnp.concatenate([z(1, 2 * Ee), dwx_ref[...]], axis=1),
    ], axis=0)
    table = _dot_t(wall_ref[...], rhs)

    idxf = jnp.clip(rl_ref[0], 0, n_run - 1).astype(jnp.float32)
    kio = jax.lax.broadcasted_iota(jnp.int32, (n_run, Nn), 0).astype(
        jnp.float32)
    ones_row = jnp.ones((1, Nn), jnp.float32)
    wb = wb_ref[...]
    for g in range(G):
        oh = (kio == idxf[g:g + 1, :]).astype(jnp.float32)
        dt_row = jnp.broadcast_to(dt_ref[0, 0, g], (1, Nn))
        oh_aug = jnp.concatenate(
            [oh, maskf_ref[0, g:g + 1, :], ones_row, dt_row], axis=0)
        xt = base_ref[0, g]
        out_ref[0, g] = _dot(wb, xt) + _dot(table, oh_aug)


def kernel(base, change_mask, run_length, delta_t, change_embed, run_embed,
           delta_W, delta_b, proj_W, proj_b):
    Bb, Tt, Nn, Dd = base.shape
    Ee = change_embed.shape[1]
    G = 200
    TG = Tt // G
    grid = (Bb * TG,)

    base_t = jnp.transpose(base, (0, 1, 3, 2))
    NB = Bb * TG
    maskf = change_mask.astype(jnp.float32).reshape(NB, G, Nn)
    rl2 = run_length.astype(jnp.int32).reshape(NB, G, Nn)
    dt3 = delta_t.astype(jnp.float32).reshape(NB, 1, G)
    wb = proj_W[:, :Dd]
    wc = proj_W[:, Dd:Dd + Ee]
    wr = proj_W[:, Dd + Ee:Dd + 2 * Ee]
    wd = proj_W[:, Dd + 2 * Ee:Dd + 3 * Ee]
    wall = jnp.concatenate([wc, wr, wd, proj_b.reshape(Dd, 1)], axis=1)
    dwx = jnp.concatenate([delta_W.reshape(1, Ee),
                           jnp.zeros((1, 1), jnp.float32)], axis=1)
    dbx = jnp.concatenate([delta_b.reshape(1, Ee),
                           jnp.ones((1, 1), jnp.float32)], axis=1)

    rep = lambda shape: pl.BlockSpec(shape, lambda i: (0, 0))
    out_t = pl.pallas_call(
        _proj_kernel,
        grid=grid,
        in_specs=[
            pl.BlockSpec((1, 1, G), lambda i: (i, 0, 0)),
            pl.BlockSpec((1, G, Nn), lambda i: (i, 0, 0)),
            pl.BlockSpec((1, G, Nn), lambda i: (i, 0, 0)),
            pl.BlockSpec((1, G, Dd, Nn),
                         lambda i: (i // TG, i % TG, 0, 0)),
            rep(change_embed.shape),
            rep(run_embed.shape),
            rep((1, Ee + 1)),
            rep((1, Ee + 1)),
            rep((Dd, Dd)),
            rep((Dd, 3 * Ee + 1)),
        ],
        out_specs=pl.BlockSpec((1, G, Dd, Nn),
                               lambda i: (i // TG, i % TG, 0, 0)),
        out_shape=jax.ShapeDtypeStruct((Bb, Tt, Dd, Nn), jnp.float32),
        compiler_params=pltpu.CompilerParams(
            dimension_semantics=("parallel",)),
    )(dt3, maskf, rl2, base_t, change_embed, run_embed, dwx, dbx,
      wb, wall)
    return jnp.transpose(out_t, (0, 1, 3, 2))


# speedup vs baseline: 7.2667x; 1.0015x over previous
import jax
import jax.numpy as jnp
from jax.experimental import pallas as pl
from jax.experimental.pallas import tpu as pltpu


def _dot_t(a, b):
    return jax.lax.dot_general(a, b, (((1,), (1,)), ((), ())),
                               preferred_element_type=jnp.float32)


def _dot(a, b):
    return jax.lax.dot_general(a, b, (((1,), (0,)), ((), ())),
                               preferred_element_type=jnp.float32)


def _proj_kernel(dt_ref, maskf_ref, rl_ref, base_ref,
                 ce_ref, re_ref, dwx_ref, dbx_ref,
                 wb_ref, wall_ref,
                 out_ref):
    _, G, Dd, Nn = base_ref.shape
    n_run = re_ref.shape[0]
    Ee = ce_ref.shape[1]

    z = lambda r, c: jnp.zeros((r, c), jnp.float32)
    re = re_ref[...]
    dce = ce_ref[1:2, :] - ce_ref[0:1, :]
    rhs = jnp.concatenate([
        jnp.concatenate([z(n_run, Ee), re, z(n_run, Ee + 1)], axis=1),
        jnp.concatenate([dce, z(1, 2 * Ee + 1)], axis=1),
        jnp.concatenate([ce_ref[0:1, :], z(1, Ee), dbx_ref[...]], axis=1),
        jnp.concatenate([z(1, 2 * Ee), dwx_ref[...]], axis=1),
    ], axis=0)
    table = _dot_t(wall_ref[...], rhs)

    idxf = jnp.clip(rl_ref[0], 0, n_run - 1).astype(jnp.float32)
    maskf = maskf_ref[0].astype(jnp.float32)
    kio = jax.lax.broadcasted_iota(jnp.int32, (n_run, Nn), 0).astype(
        jnp.float32)
    ones_row = jnp.ones((1, Nn), jnp.float32)
    wb = wb_ref[...]
    for g in range(G):
        oh = (kio == idxf[g:g + 1, :]).astype(jnp.float32)
        dt_row = jnp.broadcast_to(dt_ref[0, 0, g], (1, Nn))
        oh_aug = jnp.concatenate(
            [oh, maskf[g:g + 1, :], ones_row, dt_row], axis=0)
        xt = base_ref[0, g]
        out_ref[0, g] = _dot(wb, xt) + _dot(table, oh_aug)


def kernel(base, change_mask, run_length, delta_t, change_embed, run_embed,
           delta_W, delta_b, proj_W, proj_b):
    Bb, Tt, Nn, Dd = base.shape
    Ee = change_embed.shape[1]
    G = 200
    TG = Tt // G
    grid = (Bb * TG,)

    base_t = jnp.transpose(base, (0, 1, 3, 2))
    NB = Bb * TG
    maskf = change_mask.reshape(NB, G, Nn)
    rl2 = run_length.astype(jnp.int32).reshape(NB, G, Nn)
    dt3 = delta_t.astype(jnp.float32).reshape(NB, 1, G)
    wb = proj_W[:, :Dd]
    wc = proj_W[:, Dd:Dd + Ee]
    wr = proj_W[:, Dd + Ee:Dd + 2 * Ee]
    wd = proj_W[:, Dd + 2 * Ee:Dd + 3 * Ee]
    wall = jnp.concatenate([wc, wr, wd, proj_b.reshape(Dd, 1)], axis=1)
    dwx = jnp.concatenate([delta_W.reshape(1, Ee),
                           jnp.zeros((1, 1), jnp.float32)], axis=1)
    dbx = jnp.concatenate([delta_b.reshape(1, Ee),
                           jnp.ones((1, 1), jnp.float32)], axis=1)

    rep = lambda shape: pl.BlockSpec(shape, lambda i: (0, 0))
    out_t = pl.pallas_call(
        _proj_kernel,
        grid=grid,
        in_specs=[
            pl.BlockSpec((1, 1, G), lambda i: (i, 0, 0)),
            pl.BlockSpec((1, G, Nn), lambda i: (i, 0, 0)),
            pl.BlockSpec((1, G, Nn), lambda i: (i, 0, 0)),
            pl.BlockSpec((1, G, Dd, Nn),
                         lambda i: (i // TG, i % TG, 0, 0)),
            rep(change_embed.shape),
            rep(run_embed.shape),
            rep((1, Ee + 1)),
            rep((1, Ee + 1)),
            rep((Dd, Dd)),
            rep((Dd, 3 * Ee + 1)),
        ],
        out_specs=pl.BlockSpec((1, G, Dd, Nn),
                               lambda i: (i // TG, i % TG, 0, 0)),
        out_shape=jax.ShapeDtypeStruct((Bb, Tt, Dd, Nn), jnp.float32),
        compiler_params=pltpu.CompilerParams(
            dimension_semantics=("parallel",)),
    )(dt3, maskf, rl2, base_t, change_embed, run_embed, dwx, dbx,
      wb, wall)
    return jnp.transpose(out_t, (0, 1, 3, 2))
